# baseline (speedup 1.0000x reference)
import dataclasses
import functools

import jax
import jax.numpy as jnp
from jax.experimental import pallas as pl
from jax.experimental.pallas import tpu as pltpu
from jax.experimental.pallas import tpu_sc as plsc

TM = 256
NE = 8

_VECTOR_MESH = plsc.VectorSubcoreMesh(core_axis_name="core",
                                      subcore_axis_name="subcore")


def _sc_compiler_params():
    cp = pltpu.CompilerParams()
    if "needs_layout_passes" in pltpu.CompilerParams.__dataclass_fields__:
        cp = dataclasses.replace(cp, needs_layout_passes=False)
    return cp


_W = 128


def _chunk_idx(idx, split):
    n = idx.shape[0]
    sub = idx.reshape(n // _W, 1, 1, _W) * split
    return sub + jnp.arange(split, dtype=idx.dtype).reshape(1, split, 1, 1)


def _sc_row_gather(x, idx, split):
    n = idx.shape[0]
    h = x.shape[1]
    hc = h // split
    xs = x.reshape(x.shape[0] * split, hc)
    idx4 = _chunk_idx(idx, split)

    @functools.partial(
        pl.kernel,
        out_type=jax.ShapeDtypeStruct((n, h), x.dtype),
        mesh=_VECTOR_MESH,
        compiler_params=_sc_compiler_params(),
    )
    def gather_kernel(x_hbm, i_hbm, o_hbm):
        def body(i_vmem, o_vmem):
            pltpu.sync_copy(x_hbm.at[i_vmem.at[0, 0, 0]], o_vmem)

        pltpu.emit_pipeline(
            body,
            grid=(n // _W, split),
            in_specs=[pl.BlockSpec((1, 1, 1, _W),
                                   index_map=lambda i, j: (i, j, 0, 0))],
            out_specs=[pl.BlockSpec((_W, hc), index_map=lambda i, j: (i, j))],
            core_axis_name=("core", "subcore"),
            dimension_semantics=(pltpu.PARALLEL, pltpu.ARBITRARY),
        )(i_hbm, o_hbm)

    return gather_kernel(xs, idx4)


def _sc_row_scatter(y, idx, n_out, split):
    n, h = y.shape
    hc = h // split
    idx8 = _chunk_idx(idx, split)

    @functools.partial(
        pl.kernel,
        out_type=jax.ShapeDtypeStruct((n_out * split, hc), y.dtype),
        mesh=_VECTOR_MESH,
        compiler_params=_sc_compiler_params(),
    )
    def scatter_kernel(y_hbm, i_hbm, o_hbm):
        def body(y_vmem, i_vmem):
            pltpu.sync_copy(y_vmem, o_hbm.at[i_vmem.at[0, 0, 0]])

        pltpu.emit_pipeline(
            body,
            grid=(n // _W, split),
            in_specs=[
                pl.BlockSpec((_W, hc), index_map=lambda i, j: (i, j)),
                pl.BlockSpec((1, 1, 1, _W),
                             index_map=lambda i, j: (i, j, 0, 0)),
            ],
            out_specs=[],
            core_axis_name=("core", "subcore"),
            dimension_semantics=(pltpu.PARALLEL, pltpu.ARBITRARY),
        )(y_hbm, i_hbm)

    return scatter_kernel(y, idx8).reshape(n_out, h)


def _mlp1_body(te_ref, x_ref, wg_ref, wu_ref, o_ref):
    x = x_ref[...]
    g = jax.lax.dot_general(x, wg_ref[0], (((1,), (0,)), ((), ())),
                            preferred_element_type=jnp.float32)
    u = jax.lax.dot_general(x, wu_ref[0], (((1,), (0,)), ((), ())),
                            preferred_element_type=jnp.float32)
    o_ref[...] = ((g * jax.lax.logistic(g)) * u).astype(jnp.bfloat16)


def _mlp2_body(te_ref, h_ref, wd_ref, o_ref):
    o_ref[...] = jax.lax.dot_general(h_ref[...], wd_ref[0],
                                     (((1,), (0,)), ((), ())),
                                     preferred_element_type=jnp.float32
                                     ).astype(jnp.bfloat16)


def kernel(hidden_states, gate_proj, up_proj, down_proj, token_ids, token_to_expert):
    Bb, Ss, H = hidden_states.shape
    nE, _, EI = gate_proj.shape
    V = token_to_expert.shape[0]
    T = Bb * Ss
    P = T + nE * TM
    NT = P // TM

    flat_x = hidden_states.reshape(T, H)
    ids = jnp.clip(token_ids.reshape(-1), 0, V - 1)
    eids = jnp.take(token_to_expert, ids, axis=0)

    order = jnp.argsort(eids, stable=True)
    sorted_e = eids[order]
    counts = jnp.bincount(eids, length=nE)
    padded = ((counts + TM - 1) // TM) * TM
    start = jnp.concatenate([jnp.zeros(1, jnp.int32),
                             jnp.cumsum(padded)[:-1].astype(jnp.int32)])
    cstart = jnp.concatenate([jnp.zeros(1, jnp.int32),
                              jnp.cumsum(counts)[:-1].astype(jnp.int32)])
    pos = start[sorted_e] + (jnp.arange(T, dtype=jnp.int32) - cstart[sorted_e])
    tile_ids = jnp.arange(NT, dtype=jnp.int32) * TM
    ends = jnp.cumsum(padded).astype(jnp.int32)
    tile_expert = jnp.minimum(jnp.searchsorted(ends, tile_ids, side="right"),
                              nE - 1).astype(jnp.int32)
    total_padded = ends[-1]
    first_e = sorted_e[0]
    tile_expert = jnp.where(tile_ids < total_padded, tile_expert, first_e)
    clone_tok = order[jnp.clip(cstart[tile_expert], 0, T - 1)]
    perm = jnp.repeat(clone_tok, TM, total_repeat_length=P)
    perm = perm.at[pos].set(order)

    x32 = jax.lax.bitcast_convert_type(
        flat_x.astype(jnp.bfloat16).reshape(T, H // 2, 2), jnp.int32)
    xs32 = _sc_row_gather(x32, perm, 8)
    x_sorted = jax.lax.bitcast_convert_type(xs32, jnp.bfloat16).reshape(P, H)

    grid1 = pltpu.PrefetchScalarGridSpec(
        num_scalar_prefetch=1,
        grid=(NT,),
        in_specs=[
            pl.BlockSpec((TM, H), lambda i, te: (i, 0)),
            pl.BlockSpec((1, H, EI), lambda i, te: (te[i], 0, 0)),
            pl.BlockSpec((1, H, EI), lambda i, te: (te[i], 0, 0)),
        ],
        out_specs=pl.BlockSpec((TM, EI), lambda i, te: (i, 0)),
    )
    inter = pl.pallas_call(
        _mlp1_body, grid_spec=grid1,
        out_shape=jax.ShapeDtypeStruct((P, EI), jnp.bfloat16),
    )(tile_expert, x_sorted, gate_proj.astype(jnp.bfloat16),
      up_proj.astype(jnp.bfloat16))

    grid2 = pltpu.PrefetchScalarGridSpec(
        num_scalar_prefetch=1,
        grid=(NT,),
        in_specs=[
            pl.BlockSpec((TM, EI), lambda i, te: (i, 0)),
            pl.BlockSpec((1, EI, H), lambda i, te: (te[i], 0, 0)),
        ],
        out_specs=pl.BlockSpec((TM, H), lambda i, te: (i, 0)),
    )
    y_sorted = pl.pallas_call(
        _mlp2_body, grid_spec=grid2,
        out_shape=jax.ShapeDtypeStruct((P, H), jnp.bfloat16),
    )(tile_expert, inter, down_proj.astype(jnp.bfloat16))

    y32 = jax.lax.bitcast_convert_type(y_sorted.reshape(P, H // 2, 2),
                                       jnp.int32)
    out32 = _sc_row_scatter(y32, perm, T, 8)
    out = jax.lax.bitcast_convert_type(out32, jnp.bfloat16).reshape(T, H)
    return out.astype(jnp.float32).reshape(Bb, Ss, H)


# speedup vs baseline: 2.2565x; 2.2565x over previous
import dataclasses
import functools

import jax
import jax.numpy as jnp
from jax.experimental import pallas as pl
from jax.experimental.pallas import tpu as pltpu
from jax.experimental.pallas import tpu_sc as plsc

TM = 256
NE = 8

_VECTOR_MESH = plsc.VectorSubcoreMesh(core_axis_name="core",
                                      subcore_axis_name="subcore")


def _sc_compiler_params():
    cp = pltpu.CompilerParams()
    if "needs_layout_passes" in pltpu.CompilerParams.__dataclass_fields__:
        cp = dataclasses.replace(cp, needs_layout_passes=False)
    return cp


_W = 128


def _chunk_idx(idx, split):
    n = idx.shape[0]
    sub = idx.reshape(n // _W, 1, 1, _W) * split
    return sub + jnp.arange(split, dtype=idx.dtype).reshape(1, split, 1, 1)


def _sc_row_gather(x, idx, split):
    n = idx.shape[0]
    h = x.shape[1]
    hc = h // split
    xs = x.reshape(x.shape[0] * split, hc)
    idx4 = _chunk_idx(idx, split)

    @functools.partial(
        pl.kernel,
        out_type=jax.ShapeDtypeStruct((n, h), x.dtype),
        mesh=_VECTOR_MESH,
        compiler_params=_sc_compiler_params(),
    )
    def gather_kernel(x_hbm, i_hbm, o_hbm):
        def body(i_vmem, o_vmem):
            pltpu.sync_copy(x_hbm.at[i_vmem.at[0, 0, 0]], o_vmem)

        pltpu.emit_pipeline(
            body,
            grid=(n // _W, split),
            in_specs=[pl.BlockSpec((1, 1, 1, _W),
                                   index_map=lambda i, j: (i, j, 0, 0))],
            out_specs=[pl.BlockSpec((_W, hc), index_map=lambda i, j: (i, j))],
            core_axis_name=("core", "subcore"),
            dimension_semantics=(pltpu.PARALLEL, pltpu.ARBITRARY),
        )(i_hbm, o_hbm)

    return gather_kernel(xs, idx4)


def _sc_row_scatter(y, idx, n_out, split):
    n, h = y.shape
    hc = h // split
    idx8 = _chunk_idx(idx, split)

    @functools.partial(
        pl.kernel,
        out_type=jax.ShapeDtypeStruct((n_out * split, hc), y.dtype),
        mesh=_VECTOR_MESH,
        compiler_params=_sc_compiler_params(),
    )
    def scatter_kernel(y_hbm, i_hbm, o_hbm):
        def body(y_vmem, i_vmem):
            pltpu.sync_copy(y_vmem, o_hbm.at[i_vmem.at[0, 0, 0]])

        pltpu.emit_pipeline(
            body,
            grid=(n // _W, split),
            in_specs=[
                pl.BlockSpec((_W, hc), index_map=lambda i, j: (i, j)),
                pl.BlockSpec((1, 1, 1, _W),
                             index_map=lambda i, j: (i, j, 0, 0)),
            ],
            out_specs=[],
            core_axis_name=("core", "subcore"),
            dimension_semantics=(pltpu.PARALLEL, pltpu.ARBITRARY),
        )(y_hbm, i_hbm)

    return scatter_kernel(y, idx8).reshape(n_out, h)


def _mlp1_body(te_ref, x_ref, wg_ref, wu_ref, o_ref):
    x = x_ref[...].astype(jnp.bfloat16)
    g = jax.lax.dot_general(x, wg_ref[0], (((1,), (0,)), ((), ())),
                            preferred_element_type=jnp.float32)
    u = jax.lax.dot_general(x, wu_ref[0], (((1,), (0,)), ((), ())),
                            preferred_element_type=jnp.float32)
    o_ref[...] = ((g * jax.lax.logistic(g)) * u).astype(jnp.bfloat16)


def _mlp2_body(te_ref, h_ref, wd_ref, o_ref):
    o_ref[...] = jax.lax.dot_general(h_ref[...], wd_ref[0],
                                     (((1,), (0,)), ((), ())),
                                     preferred_element_type=jnp.float32)


def kernel(hidden_states, gate_proj, up_proj, down_proj, token_ids, token_to_expert):
    Bb, Ss, H = hidden_states.shape
    nE, _, EI = gate_proj.shape
    V = token_to_expert.shape[0]
    T = Bb * Ss
    P = T + nE * TM
    NT = P // TM

    flat_x = hidden_states.reshape(T, H)
    ids = jnp.clip(token_ids.reshape(-1), 0, V - 1)
    eids = jnp.take(token_to_expert, ids, axis=0)

    order = jnp.argsort(eids, stable=True)
    sorted_e = eids[order]
    counts = jnp.bincount(eids, length=nE)
    padded = ((counts + TM - 1) // TM) * TM
    start = jnp.concatenate([jnp.zeros(1, jnp.int32),
                             jnp.cumsum(padded)[:-1].astype(jnp.int32)])
    cstart = jnp.concatenate([jnp.zeros(1, jnp.int32),
                              jnp.cumsum(counts)[:-1].astype(jnp.int32)])
    pos = start[sorted_e] + (jnp.arange(T, dtype=jnp.int32) - cstart[sorted_e])
    tile_ids = jnp.arange(NT, dtype=jnp.int32) * TM
    ends = jnp.cumsum(padded).astype(jnp.int32)
    tile_expert = jnp.minimum(jnp.searchsorted(ends, tile_ids, side="right"),
                              nE - 1).astype(jnp.int32)
    total_padded = ends[-1]
    first_e = sorted_e[0]
    tile_expert = jnp.where(tile_ids < total_padded, tile_expert, first_e)
    clone_tok = order[jnp.clip(cstart[tile_expert], 0, T - 1)]
    perm = jnp.repeat(clone_tok, TM, total_repeat_length=P)
    perm = perm.at[pos].set(order)

    x_sorted = _sc_row_gather(flat_x, perm, 8)

    grid1 = pltpu.PrefetchScalarGridSpec(
        num_scalar_prefetch=1,
        grid=(NT,),
        in_specs=[
            pl.BlockSpec((TM, H), lambda i, te: (i, 0)),
            pl.BlockSpec((1, H, EI), lambda i, te: (te[i], 0, 0)),
            pl.BlockSpec((1, H, EI), lambda i, te: (te[i], 0, 0)),
        ],
        out_specs=pl.BlockSpec((TM, EI), lambda i, te: (i, 0)),
    )
    inter = pl.pallas_call(
        _mlp1_body, grid_spec=grid1,
        out_shape=jax.ShapeDtypeStruct((P, EI), jnp.bfloat16),
    )(tile_expert, x_sorted, gate_proj.astype(jnp.bfloat16),
      up_proj.astype(jnp.bfloat16))

    grid2 = pltpu.PrefetchScalarGridSpec(
        num_scalar_prefetch=1,
        grid=(NT,),
        in_specs=[
            pl.BlockSpec((TM, EI), lambda i, te: (i, 0)),
            pl.BlockSpec((1, EI, H), lambda i, te: (te[i], 0, 0)),
        ],
        out_specs=pl.BlockSpec((TM, H), lambda i, te: (i, 0)),
    )
    y_sorted = pl.pallas_call(
        _mlp2_body, grid_spec=grid2,
        out_shape=jax.ShapeDtypeStruct((P, H), jnp.float32),
    )(tile_expert, inter, down_proj.astype(jnp.bfloat16))

    out = _sc_row_scatter(y_sorted, perm, T, 8)
    return out.reshape(Bb, Ss, H)


# speedup vs baseline: 2.2688x; 1.0055x over previous
import dataclasses
import functools

import jax
import jax.numpy as jnp
from jax.experimental import pallas as pl
from jax.experimental.pallas import tpu as pltpu
from jax.experimental.pallas import tpu_sc as plsc

TM = 256
NE = 8

_VECTOR_MESH = plsc.VectorSubcoreMesh(core_axis_name="core",
                                      subcore_axis_name="subcore")


def _sc_compiler_params():
    cp = pltpu.CompilerParams()
    if "needs_layout_passes" in pltpu.CompilerParams.__dataclass_fields__:
        cp = dataclasses.replace(cp, needs_layout_passes=False)
    return cp


_W = 128


def _chunk_idx(idx, split):
    n = idx.shape[0]
    sub = idx.reshape(n // _W, 1, 1, _W) * split
    return sub + jnp.arange(split, dtype=idx.dtype).reshape(1, split, 1, 1)


def _sc_row_gather(x, idx, split):
    n = idx.shape[0]
    h = x.shape[1]
    hc = h // split
    xs = x.reshape(x.shape[0] * split, hc)
    idx4 = _chunk_idx(idx, split)

    @functools.partial(
        pl.kernel,
        out_type=jax.ShapeDtypeStruct((n, h), x.dtype),
        mesh=_VECTOR_MESH,
        compiler_params=_sc_compiler_params(),
    )
    def gather_kernel(x_hbm, i_hbm, o_hbm):
        def body(i_vmem, o_vmem):
            pltpu.sync_copy(x_hbm.at[i_vmem.at[0, 0, 0]], o_vmem)

        pltpu.emit_pipeline(
            body,
            grid=(n // _W, split),
            in_specs=[pl.BlockSpec((1, 1, 1, _W),
                                   index_map=lambda i, j: (i, j, 0, 0))],
            out_specs=[pl.BlockSpec((_W, hc), index_map=lambda i, j: (i, j))],
            core_axis_name=("core", "subcore"),
            dimension_semantics=(pltpu.PARALLEL, pltpu.PARALLEL),
        )(i_hbm, o_hbm)

    return gather_kernel(xs, idx4)


def _sc_row_scatter(y, idx, n_out, split):
    n, h = y.shape
    hc = h // split
    idx8 = _chunk_idx(idx, split)

    @functools.partial(
        pl.kernel,
        out_type=jax.ShapeDtypeStruct((n_out * split, hc), y.dtype),
        mesh=_VECTOR_MESH,
        compiler_params=_sc_compiler_params(),
    )
    def scatter_kernel(y_hbm, i_hbm, o_hbm):
        def body(y_vmem, i_vmem):
            pltpu.sync_copy(y_vmem, o_hbm.at[i_vmem.at[0, 0, 0]])

        pltpu.emit_pipeline(
            body,
            grid=(n // _W, split),
            in_specs=[
                pl.BlockSpec((_W, hc), index_map=lambda i, j: (i, j)),
                pl.BlockSpec((1, 1, 1, _W),
                             index_map=lambda i, j: (i, j, 0, 0)),
            ],
            out_specs=[],
            core_axis_name=("core", "subcore"),
            dimension_semantics=(pltpu.PARALLEL, pltpu.PARALLEL),
        )(y_hbm, i_hbm)

    return scatter_kernel(y, idx8).reshape(n_out, h)


def _mlp1_body(te_ref, x_ref, wg_ref, wu_ref, o_ref):
    x = x_ref[...].astype(jnp.bfloat16)
    g = jax.lax.dot_general(x, wg_ref[0], (((1,), (0,)), ((), ())),
                            preferred_element_type=jnp.float32)
    u = jax.lax.dot_general(x, wu_ref[0], (((1,), (0,)), ((), ())),
                            preferred_element_type=jnp.float32)
    o_ref[...] = ((g * jax.lax.logistic(g)) * u).astype(jnp.bfloat16)


def _mlp2_body(te_ref, h_ref, wd_ref, o_ref):
    o_ref[...] = jax.lax.dot_general(h_ref[...], wd_ref[0],
                                     (((1,), (0,)), ((), ())),
                                     preferred_element_type=jnp.float32)


def kernel(hidden_states, gate_proj, up_proj, down_proj, token_ids, token_to_expert):
    Bb, Ss, H = hidden_states.shape
    nE, _, EI = gate_proj.shape
    V = token_to_expert.shape[0]
    T = Bb * Ss
    P = T + nE * TM
    NT = P // TM

    flat_x = hidden_states.reshape(T, H)
    ids = jnp.clip(token_ids.reshape(-1), 0, V - 1)
    eids = jnp.take(token_to_expert, ids, axis=0)

    order = jnp.argsort(eids, stable=True)
    sorted_e = eids[order]
    counts = jnp.bincount(eids, length=nE)
    padded = ((counts + TM - 1) // TM) * TM
    start = jnp.concatenate([jnp.zeros(1, jnp.int32),
                             jnp.cumsum(padded)[:-1].astype(jnp.int32)])
    cstart = jnp.concatenate([jnp.zeros(1, jnp.int32),
                              jnp.cumsum(counts)[:-1].astype(jnp.int32)])
    pos = start[sorted_e] + (jnp.arange(T, dtype=jnp.int32) - cstart[sorted_e])
    tile_ids = jnp.arange(NT, dtype=jnp.int32) * TM
    ends = jnp.cumsum(padded).astype(jnp.int32)
    tile_expert = jnp.minimum(jnp.searchsorted(ends, tile_ids, side="right"),
                              nE - 1).astype(jnp.int32)
    total_padded = ends[-1]
    first_e = sorted_e[0]
    tile_expert = jnp.where(tile_ids < total_padded, tile_expert, first_e)
    clone_tok = order[jnp.clip(cstart[tile_expert], 0, T - 1)]
    perm = jnp.repeat(clone_tok, TM, total_repeat_length=P)
    perm = perm.at[pos].set(order)

    x_sorted = _sc_row_gather(flat_x, perm, 8)

    grid1 = pltpu.PrefetchScalarGridSpec(
        num_scalar_prefetch=1,
        grid=(NT,),
        in_specs=[
            pl.BlockSpec((TM, H), lambda i, te: (i, 0)),
            pl.BlockSpec((1, H, EI), lambda i, te: (te[i], 0, 0)),
            pl.BlockSpec((1, H, EI), lambda i, te: (te[i], 0, 0)),
        ],
        out_specs=pl.BlockSpec((TM, EI), lambda i, te: (i, 0)),
    )
    inter = pl.pallas_call(
        _mlp1_body, grid_spec=grid1,
        out_shape=jax.ShapeDtypeStruct((P, EI), jnp.bfloat16),
    )(tile_expert, x_sorted, gate_proj.astype(jnp.bfloat16),
      up_proj.astype(jnp.bfloat16))

    grid2 = pltpu.PrefetchScalarGridSpec(
        num_scalar_prefetch=1,
        grid=(NT,),
        in_specs=[
            pl.BlockSpec((TM, EI), lambda i, te: (i, 0)),
            pl.BlockSpec((1, EI, H), lambda i, te: (te[i], 0, 0)),
        ],
        out_specs=pl.BlockSpec((TM, H), lambda i, te: (i, 0)),
    )
    y_sorted = pl.pallas_call(
        _mlp2_body, grid_spec=grid2,
        out_shape=jax.ShapeDtypeStruct((P, H), jnp.float32),
    )(tile_expert, inter, down_proj.astype(jnp.bfloat16))

    out = _sc_row_scatter(y_sorted, perm, T, 8)
    return out.reshape(Bb, Ss, H)
